# baseline (speedup 1.0000x reference)
import functools

import jax
import jax.numpy as jnp
from jax import lax
from jax.experimental import pallas as pl
from jax.experimental.pallas import tpu as pltpu
from jax.experimental.pallas import tpu_sc as plsc

B = 16384
NF = 26
NV = 4
VOCAB = 100000
L = 50
D = 32

NC = 2
NS = 16
NW = NC * NS

CB = 8
CHUNKS = B // (NW * CB)
NOUT = NF + NV
ROWS_F = CB * NF
ROWS_V = CB * NV * L
POOLS = CB * NV
OUT_R = CB * NOUT

FW = 208
VW = 1600
NFB = ROWS_F // FW
NVB = ROWS_V // VW

_mesh = plsc.VectorSubcoreMesh(core_axis_name="c", subcore_axis_name="s")


@functools.partial(
    pl.kernel,
    out_type=jax.ShapeDtypeStruct((B * NOUT, D), jnp.float32),
    mesh=_mesh,
    scratch_types=[
        [pltpu.VMEM((NFB, FW), jnp.int32)] * 2,
        [pltpu.VMEM((NVB, VW), jnp.int32)] * 2,
        [pltpu.VMEM((ROWS_F, D), jnp.float32)] * 2,
        [pltpu.VMEM((ROWS_V, D), jnp.float32)] * 2,
        [pltpu.VMEM((POOLS, D), jnp.float32)] * 2,
        pltpu.SemaphoreType.DMA,
        pltpu.SemaphoreType.DMA,
        pltpu.SemaphoreType.DMA,
    ],
    compiler_params=pltpu.CompilerParams(use_tc_tiling_on_sc=False),
)
def _emb(wf, wv, idxf, idxv, out,
         if_v, iv_v, fb, vb, sv, sem_i, sem_g, sem_o):
    sid = lax.axis_index("s")
    wid = sid * NC + lax.axis_index("c")
    base = wid * CHUNKS

    def stage_idx(g, p):
        pltpu.async_copy(idxf.at[pl.ds(g * NFB, NFB)], if_v[p], sem_i)
        pltpu.async_copy(idxv.at[pl.ds(g * NVB, NVB)], iv_v[p], sem_i)

    def drain_idx(p):
        pltpu.make_async_copy(idxf.at[pl.ds(0, NFB)], if_v[p], sem_i).wait()
        pltpu.make_async_copy(idxv.at[pl.ds(0, NVB)], iv_v[p], sem_i).wait()

    def fire_gathers(p):
        for r in range(NFB):
            pltpu.async_copy(
                wf.at[if_v[p].at[r]], fb[p].at[pl.ds(r * FW, FW)], sem_g)
        for r in range(NVB):
            pltpu.async_copy(
                wv.at[iv_v[p].at[r]], vb[p].at[pl.ds(r * VW, VW)], sem_g)

    def drain_gathers(p):
        for r in range(NFB):
            pltpu.make_async_copy(
                wf.at[if_v[p].at[r]], fb[p].at[pl.ds(r * FW, FW)],
                sem_g).wait()
        for r in range(NVB):
            pltpu.make_async_copy(
                wv.at[iv_v[p].at[r]], vb[p].at[pl.ds(r * VW, VW)],
                sem_g).wait()

    def pool_compute(p):
        vb_p, sv_p = vb[p], sv[p]

        def pool_body(k, carry):
            row = k * L
            z = jnp.zeros((16,), jnp.float32)
            acc = [[z] * 4, [z] * 4]
            for j in range(L):
                lane = j % 4
                acc[0][lane] = acc[0][lane] + vb_p[row + j, pl.ds(0, 16)]
                acc[1][lane] = acc[1][lane] + vb_p[row + j, pl.ds(16, 16)]
            for h, off in ((0, 0), (1, 16)):
                s = (acc[h][0] + acc[h][1]) + (acc[h][2] + acc[h][3])
                sv_p[k, pl.ds(off, 16)] = s * (1.0 / L)
            return carry

        lax.fori_loop(0, POOLS, pool_body, 0)

    def fire_out(g, p):
        for b in range(CB):
            pltpu.async_copy(
                fb[p].at[pl.ds(b * NF, NF)],
                out.at[pl.ds(g * OUT_R + b * NOUT, NF)], sem_o)
            pltpu.async_copy(
                sv[p].at[pl.ds(b * NV, NV)],
                out.at[pl.ds(g * OUT_R + b * NOUT + NF, NV)], sem_o)

    def drain_out(p):
        for b in range(CB):
            pltpu.make_async_copy(
                fb[p].at[pl.ds(b * NF, NF)],
                out.at[pl.ds(b * NOUT, NF)], sem_o).wait()
            pltpu.make_async_copy(
                sv[p].at[pl.ds(b * NV, NV)],
                out.at[pl.ds(b * NOUT + NF, NV)], sem_o).wait()

    def step(i, p, drain_prev, prefetch):
        g = base + i
        if prefetch:
            stage_idx(g + 1, 1 - p)
        drain_gathers(p)
        if drain_prev:
            drain_out(1 - p)
        if prefetch:
            drain_idx(1 - p)
            fire_gathers(1 - p)
        pool_compute(p)
        fire_out(g, p)

    stage_idx(base, 0)
    drain_idx(0)
    fire_gathers(0)

    step(0, 0, False, True)
    step(1, 1, True, True)

    def body(k, carry):
        step(2 * k, 0, True, True)
        step(2 * k + 1, 1, True, True)
        return carry

    lax.fori_loop(1, CHUNKS // 2 - 1, body, 0)

    step(CHUNKS - 2, 0, True, True)
    step(CHUNKS - 1, 1, True, False)
    drain_out(1)


def kernel(x_fix, x_varlen, W_fix, W_var):
    wf = W_fix.reshape(NF * VOCAB, D)
    wv = W_var.reshape(NV * VOCAB, D)
    offs_f = (jnp.arange(NF, dtype=jnp.int32) * VOCAB)[None, :]
    offs_v = (jnp.arange(NV, dtype=jnp.int32) * VOCAB)[None, :, None]
    idxf = (x_fix.astype(jnp.int32) + offs_f).reshape(B * NF // FW, FW)
    idxv = (x_varlen.astype(jnp.int32) + offs_v).reshape(B * NV * L // VW, VW)
    out = _emb(wf, wv, idxf, idxv)
    return out.reshape(B, NOUT * D)
